# baseline (speedup 1.0000x reference)
import functools

import jax
import jax.numpy as jnp
from jax import lax
from jax.experimental import pallas as pl
from jax.experimental.pallas import tpu as pltpu
import jax.experimental.pallas.tpu_sc as plsc

B = 16384
D = 32
DP = D // 2
L = 16
NC = 2
NS = 16
NW = NC * NS
BPW = B // NW

def _fm_body(uid_hbm, iid_hbm, uf_hbm, vf_hbm, ub_hbm, ib_hbm, gb_hbm,
             pred_hbm, cvr_hbm,
             uid_v, iid_v, urows, vrows, ubias_v, ibias_v, gb_v,
             pred_v, cvr_v, sem):
  wid = lax.axis_index("s") * NC + lax.axis_index("c")
  base = wid * BPW

  pltpu.sync_copy(uid_hbm.at[pl.ds(base, BPW)], uid_v)
  pltpu.sync_copy(iid_hbm.at[pl.ds(base, BPW)], iid_v)
  pltpu.sync_copy(gb_hbm, gb_v)

  cu = pltpu.async_copy(uf_hbm.at[uid_v], urows, sem)
  cv = pltpu.async_copy(vf_hbm.at[iid_v], vrows, sem)
  cub = pltpu.async_copy(ub_hbm.at[uid_v], ubias_v, sem)
  cib = pltpu.async_copy(ib_hbm.at[iid_v], ibias_v, sem)
  cu.wait()
  cv.wait()
  cub.wait()
  cib.wait()

  gb = gb_v[...]

  def unpack(w):
    lo = plsc.bitcast(lax.shift_left(w, 16), jnp.float32)
    hi = plsc.bitcast(lax.bitwise_and(w, jnp.int32(-65536)), jnp.float32)
    return lo, hi

  def group(g, carry):
    acc = ubias_v[pl.ds(g * L, L)] + ibias_v[pl.ds(g * L, L)] + gb
    rows = lax.broadcasted_iota(jnp.int32, (L,), 0) + g * L
    for dp in range(DP):
      cols = jnp.full((L,), dp, jnp.int32)
      wu = plsc.load_gather(urows, [rows, cols])
      wv = plsc.load_gather(vrows, [rows, cols])
      ulo, uhi = unpack(wu)
      vlo, vhi = unpack(wv)
      acc = acc + ulo * vlo + uhi * vhi
    pred_v[pl.ds(g * L, L)] = acc
    cvr_v[pl.ds(g * L, L)] = 1.0 / (1.0 + jnp.exp(-acc))
    return carry

  lax.fori_loop(0, BPW // L, group, 0)

  pltpu.sync_copy(pred_v, pred_hbm.at[pl.ds(base, BPW)])
  pltpu.sync_copy(cvr_v, cvr_hbm.at[pl.ds(base, BPW)])


def _pack_table(t):
  t16 = t.astype(jnp.bfloat16).reshape(t.shape[0], t.shape[1] // 2, 2)
  return jax.lax.bitcast_convert_type(t16, jnp.int32)


def kernel(user_id, item_id, user_factors, item_factors, user_bias,
           item_bias, global_bias):
  gb16 = jnp.broadcast_to(global_bias.astype(jnp.float32), (L,))
  mesh = plsc.VectorSubcoreMesh(core_axis_name="c", subcore_axis_name="s")

  fm = pl.kernel(
      _fm_body,
      out_type=(
          jax.ShapeDtypeStruct((B,), jnp.float32),
          jax.ShapeDtypeStruct((B,), jnp.float32),
      ),
      mesh=mesh,
      compiler_params=pltpu.CompilerParams(
          needs_layout_passes=False, use_tc_tiling_on_sc=False),
      scratch_types=[
          pltpu.VMEM((BPW,), jnp.int32),
          pltpu.VMEM((BPW,), jnp.int32),
          pltpu.VMEM((BPW, DP), jnp.int32),
          pltpu.VMEM((BPW, DP), jnp.int32),
          pltpu.VMEM((BPW,), jnp.float32),
          pltpu.VMEM((BPW,), jnp.float32),
          pltpu.VMEM((L,), jnp.float32),
          pltpu.VMEM((BPW,), jnp.float32),
          pltpu.VMEM((BPW,), jnp.float32),
          pltpu.SemaphoreType.DMA,
      ],
  )
  pred, cvr = fm(user_id.astype(jnp.int32), item_id.astype(jnp.int32),
                 _pack_table(user_factors), _pack_table(item_factors),
                 user_bias, item_bias, gb16)
  return pred, cvr


# speedup vs baseline: 2.1510x; 2.1510x over previous
import functools

import jax
import jax.numpy as jnp
from jax import lax
from jax.experimental import pallas as pl
from jax.experimental.pallas import tpu as pltpu
import jax.experimental.pallas.tpu_sc as plsc

B = 16384
D = 32
G = 4
L = 16
NC = 2
NS = 16
NW = NC * NS
BPW = B // NW
CH = 256


def _fm_body(uid_hbm, iid_hbm, ufg_hbm, vfg_hbm, ub_hbm, ib_hbm, gb_hbm,
             pred_hbm, cvr_hbm,
             uid_v, iid_v, ugidx_v, igidx_v, ugrp, vgrp, ubias_v, ibias_v,
             gb_v, pred_v, cvr_v, sem, bsem):
  wid = lax.axis_index("s") * NC + lax.axis_index("c")
  base = wid * BPW

  pltpu.sync_copy(uid_hbm.at[pl.ds(base, BPW)], uid_v)
  pltpu.sync_copy(iid_hbm.at[pl.ds(base, BPW)], iid_v)
  pltpu.sync_copy(gb_hbm, gb_v)

  def mkidx(k, carry):
    ugidx_v[pl.ds(k * L, L)] = lax.shift_right_logical(
        uid_v[pl.ds(k * L, L)], 2)
    igidx_v[pl.ds(k * L, L)] = lax.shift_right_logical(
        iid_v[pl.ds(k * L, L)], 2)
    return carry

  lax.fori_loop(0, BPW // L, mkidx, 0)

  cub = pltpu.async_copy(ub_hbm.at[uid_v], ubias_v, bsem)
  cib = pltpu.async_copy(ib_hbm.at[iid_v], ibias_v, bsem)
  cub.wait()
  cib.wait()

  gb = gb_v[...]

  def chunk(h, carry):
    cu = pltpu.async_copy(ufg_hbm.at[ugidx_v.at[pl.ds(h * CH, CH)]], ugrp, sem)
    cv = pltpu.async_copy(vfg_hbm.at[igidx_v.at[pl.ds(h * CH, CH)]], vgrp, sem)
    cu.wait()
    cv.wait()

    def group(k, c):
      j0 = h * CH + k * L
      acc = ubias_v[pl.ds(j0, L)] + ibias_v[pl.ds(j0, L)] + gb
      rows = lax.broadcasted_iota(jnp.int32, (L,), 0) + k * L
      ucols0 = lax.bitwise_and(uid_v[pl.ds(j0, L)], jnp.int32(G - 1)) * D
      vcols0 = lax.bitwise_and(iid_v[pl.ds(j0, L)], jnp.int32(G - 1)) * D
      for d in range(D):
        au = plsc.load_gather(ugrp, [rows, ucols0 + d])
        av = plsc.load_gather(vgrp, [rows, vcols0 + d])
        acc = acc + au * av
      pred_v[pl.ds(j0, L)] = acc
      cvr_v[pl.ds(j0, L)] = 1.0 / (1.0 + jnp.exp(-acc))
      return c

    lax.fori_loop(0, CH // L, group, 0)
    return carry

  lax.fori_loop(0, BPW // CH, chunk, 0)

  pltpu.sync_copy(pred_v, pred_hbm.at[pl.ds(base, BPW)])
  pltpu.sync_copy(cvr_v, cvr_hbm.at[pl.ds(base, BPW)])


def kernel(user_id, item_id, user_factors, item_factors, user_bias,
           item_bias, global_bias):
  gb16 = jnp.broadcast_to(global_bias.astype(jnp.float32), (L,))
  mesh = plsc.VectorSubcoreMesh(core_axis_name="c", subcore_axis_name="s")

  fm = pl.kernel(
      _fm_body,
      out_type=(
          jax.ShapeDtypeStruct((B,), jnp.float32),
          jax.ShapeDtypeStruct((B,), jnp.float32),
      ),
      mesh=mesh,
      compiler_params=pltpu.CompilerParams(
          needs_layout_passes=False, use_tc_tiling_on_sc=True),
      scratch_types=[
          pltpu.VMEM((BPW,), jnp.int32),
          pltpu.VMEM((BPW,), jnp.int32),
          pltpu.VMEM((BPW,), jnp.int32),
          pltpu.VMEM((BPW,), jnp.int32),
          pltpu.VMEM((CH, 128), jnp.float32),
          pltpu.VMEM((CH, 128), jnp.float32),
          pltpu.VMEM((BPW,), jnp.float32),
          pltpu.VMEM((BPW,), jnp.float32),
          pltpu.VMEM((L,), jnp.float32),
          pltpu.VMEM((BPW,), jnp.float32),
          pltpu.VMEM((BPW,), jnp.float32),
          pltpu.SemaphoreType.DMA,
          pltpu.SemaphoreType.DMA,
      ],
  )
  pred, cvr = fm(user_id.astype(jnp.int32), item_id.astype(jnp.int32),
                 user_factors.reshape(250000, 128),
                 item_factors.reshape(250000, 128),
                 user_bias, item_bias, gb16)
  return pred, cvr


# speedup vs baseline: 2.1711x; 1.0094x over previous
import functools

import jax
import jax.numpy as jnp
from jax import lax
from jax.experimental import pallas as pl
from jax.experimental.pallas import tpu as pltpu
import jax.experimental.pallas.tpu_sc as plsc

B = 16384
D = 32
L = 16
NC = 2
NS = 16
NW = NC * NS
BPW = B // NW


def _fm_body(uid_hbm, iid_hbm, uf_hbm, vf_hbm, ub_hbm, ib_hbm, gb_hbm,
             pred_hbm, cvr_hbm,
             uid_v, iid_v, urows, vrows, ubias_v, ibias_v, gb_v,
             pred_v, cvr_v, sem):
  wid = lax.axis_index("s") * NC + lax.axis_index("c")
  base = wid * BPW

  pltpu.sync_copy(uid_hbm.at[pl.ds(base, BPW)], uid_v)
  pltpu.sync_copy(iid_hbm.at[pl.ds(base, BPW)], iid_v)
  pltpu.sync_copy(gb_hbm, gb_v)

  cu = pltpu.async_copy(uf_hbm.at[uid_v], urows, sem)
  cv = pltpu.async_copy(vf_hbm.at[iid_v], vrows, sem)
  cub = pltpu.async_copy(ub_hbm.at[uid_v], ubias_v, sem)
  cib = pltpu.async_copy(ib_hbm.at[iid_v], ibias_v, sem)
  cu.wait()
  cv.wait()
  cub.wait()
  cib.wait()

  gb = gb_v[...]

  def group(g, carry):
    acc = ubias_v[pl.ds(g * L, L)] + ibias_v[pl.ds(g * L, L)] + gb
    rows = lax.broadcasted_iota(jnp.int32, (L,), 0) + g * L
    for d in range(D):
      cols = jnp.full((L,), d, jnp.int32)
      au = plsc.load_gather(urows, [rows, cols])
      av = plsc.load_gather(vrows, [rows, cols])
      acc = acc + au * av
    pred_v[pl.ds(g * L, L)] = acc
    cvr_v[pl.ds(g * L, L)] = 1.0 / (1.0 + jnp.exp(-acc))
    return carry

  lax.fori_loop(0, BPW // L, group, 0)

  pltpu.sync_copy(pred_v, pred_hbm.at[pl.ds(base, BPW)])
  pltpu.sync_copy(cvr_v, cvr_hbm.at[pl.ds(base, BPW)])


def kernel(user_id, item_id, user_factors, item_factors, user_bias,
           item_bias, global_bias):
  gb16 = jnp.broadcast_to(global_bias.astype(jnp.float32), (L,))
  mesh = plsc.VectorSubcoreMesh(core_axis_name="c", subcore_axis_name="s")

  fm = pl.kernel(
      _fm_body,
      out_type=(
          jax.ShapeDtypeStruct((B,), jnp.float32),
          jax.ShapeDtypeStruct((B,), jnp.float32),
      ),
      mesh=mesh,
      compiler_params=pltpu.CompilerParams(
          needs_layout_passes=False, use_tc_tiling_on_sc=False),
      scratch_types=[
          pltpu.VMEM((BPW,), jnp.int32),
          pltpu.VMEM((BPW,), jnp.int32),
          pltpu.VMEM((BPW, D), jnp.float32),
          pltpu.VMEM((BPW, D), jnp.float32),
          pltpu.VMEM((BPW,), jnp.float32),
          pltpu.VMEM((BPW,), jnp.float32),
          pltpu.VMEM((L,), jnp.float32),
          pltpu.VMEM((BPW,), jnp.float32),
          pltpu.VMEM((BPW,), jnp.float32),
          pltpu.SemaphoreType.DMA,
      ],
  )
  pred, cvr = fm(user_id.astype(jnp.int32), item_id.astype(jnp.int32),
                 user_factors, item_factors, user_bias, item_bias, gb16)
  return pred, cvr
